# baseline (speedup 1.0000x reference)
import jax
import jax.numpy as jnp
from jax.experimental import pallas as pl

B = 16384
K = 64
BLK = 2048


def _body(gu_ref, gi_ref, bu_ref, bi_ref, mu_ref, out_ref):
    prod = gu_ref[...] * gi_ref[...]
    s = jnp.sum(prod, axis=1)
    out_ref[...] = s + bu_ref[:, 0] + bi_ref[:, 0] + mu_ref[0, 0]


def kernel(gu, gi, bu, bi, Mu):
    grid = (B // BLK,)
    out = pl.pallas_call(
        _body,
        grid=grid,
        in_specs=[
            pl.BlockSpec((BLK, K), lambda i: (i, 0)),
            pl.BlockSpec((BLK, K), lambda i: (i, 0)),
            pl.BlockSpec((BLK, 1), lambda i: (i, 0)),
            pl.BlockSpec((BLK, 1), lambda i: (i, 0)),
            pl.BlockSpec((1, 1), lambda i: (0, 0)),
        ],
        out_specs=pl.BlockSpec((BLK,), lambda i: (i,)),
        out_shape=jax.ShapeDtypeStruct((B,), jnp.float32),
    )(gu, gi, bu, bi, Mu)
    return out
